# baseline (speedup 1.0000x reference)
import functools

import jax
import jax.numpy as jnp
from jax import lax
from jax.experimental import pallas as pl
from jax.experimental.pallas import tpu as pltpu
from jax.experimental.pallas import tpu_sc as plsc

_NC = 2
_NS = 16
_K = 40
_NB = 6


def _dense_body(x_ref, w0_ref, b0_ref, w1_ref, b1_ref, msg_ref):
    x = x_ref[...]
    h = jnp.dot(x, w0_ref[...], preferred_element_type=jnp.float32) + b0_ref[...]
    msg_ref[...] = jnp.dot(h, w1_ref[...], preferred_element_type=jnp.float32) + b1_ref[...]


def _dense_call(X, W0, b0, W1, b1, block_n):
    n, d = X.shape
    grid = n // block_n
    return pl.pallas_call(
        _dense_body,
        grid=(grid,),
        in_specs=[
            pl.BlockSpec((block_n, d), lambda i: (i, 0)),
            pl.BlockSpec(W0.shape, lambda i: (0, 0)),
            pl.BlockSpec(b0.shape, lambda i: (0, 0)),
            pl.BlockSpec(W1.shape, lambda i: (0, 0)),
            pl.BlockSpec(b1.shape, lambda i: (0, 0)),
        ],
        out_specs=pl.BlockSpec((block_n, d), lambda i: (i, 0)),
        out_shape=jax.ShapeDtypeStruct((n, d), jnp.float32),
    )(X, W0, b0, W1, b1)


def _sc_body(n, e, d, nchunk, xmsg_hbm, ra1_hbm, rb1_hbm,
             zeros_hbm, out_hbm, acc, gidx, sidx, rows, gsems, ssems):
    epw = nchunk * _K
    rpt = (n // _NS) // 8 * 8
    rem = n - _NS * rpt
    c = lax.axis_index("c")
    s = lax.axis_index("s")
    w = c * _NS + s
    r0 = s * rpt
    sets = tuple((rows[p], gsems[p], ssems[p]) for p in range(_NB))

    def one_direction(gi, si):
        def gather(i, p):
            row, gsem, _ = sets[p]
            pltpu.async_copy(xmsg_hbm.at[gi.at[pl.ds(i * _K, _K)]], row,
                             gsem)

        def wait_gather(i, p):
            row, gsem, _ = sets[p]
            pltpu.make_async_copy(xmsg_hbm.at[gi.at[pl.ds(i * _K, _K)]],
                                  row, gsem).wait()

        def scatter(i, p):
            row, _, ssem = sets[p]
            pltpu.async_copy(row, acc.at[si.at[pl.ds(i * _K, _K)]], ssem,
                             add=True)

        def wait_scatter(i, p):
            row, _, ssem = sets[p]
            pltpu.make_async_copy(row, acc.at[si.at[pl.ds(i * _K, _K)]],
                                  ssem).wait()

        def step(i, p, prefetch, wait_prev=True):
            wait_gather(i, p)
            scatter(i, p)
            if prefetch:
                pm1 = (p + _NB - 1) % _NB
                if wait_prev:
                    wait_scatter(i - 1, pm1)
                gather(i + _NB - 1, pm1)

        for j in range(_NB - 1):
            gather(j, j)
        for i in range(_NB - 1):
            step(i, i, True, wait_prev=(i >= 1))
        lo = _NB - 1
        hi = nchunk - _NB
        iters = (hi - lo + 1) // _NB

        def block(t, carry):
            i0 = lo + _NB * t
            for k in range(_NB):
                step(i0 + k, (lo + k) % _NB, True)
            return carry

        lax.fori_loop(0, iters, block, 0)
        for i in range(lo + iters * _NB, hi + 1):
            step(i, i % _NB, True)
        for i in range(hi + 1, nchunk):
            step(i, i % _NB, False)
        for j in range(nchunk - _NB, nchunk):
            wait_scatter(j, j % _NB)

    psem = gsems[0]
    cz = pltpu.async_copy(zeros_hbm.at[pl.ds(r0, rpt)],
                          acc.at[pl.ds(r0, rpt)], psem)
    ca = pltpu.async_copy(ra1_hbm.at[pl.ds(w * epw, epw)], gidx, psem)
    cb = pltpu.async_copy(rb1_hbm.at[pl.ds(w * epw, epw)], sidx, psem)
    if rem:
        @pl.when(s == _NS - 1)
        def _zero_rem():
            pltpu.sync_copy(zeros_hbm.at[pl.ds(_NS * rpt, rem)],
                            acc.at[pl.ds(_NS * rpt, rem)])
    cz.wait()
    ca.wait()
    cb.wait()
    plsc.subcore_barrier()

    one_direction(gidx, sidx)
    one_direction(sidx, gidx)

    plsc.subcore_barrier()
    pltpu.sync_copy(acc.at[pl.ds(r0, rpt)], out_hbm.at[pl.ds(c * n + r0, rpt)])
    if rem:
        @pl.when(s == _NS - 1)
        def _out_rem():
            pltpu.sync_copy(acc.at[pl.ds(_NS * rpt, rem)],
                            out_hbm.at[pl.ds(c * n + _NS * rpt, rem)])


def _sc_call(msg, ref_a, ref_b, zeros):
    n, d = msg.shape
    e = ref_a.shape[0]
    nw = _NC * _NS
    nchunk = e // (nw * _K)
    epw = nchunk * _K
    mesh = plsc.VectorSubcoreMesh(core_axis_name="c", subcore_axis_name="s")
    run = pl.kernel(
        functools.partial(_sc_body, n, e, d, nchunk),
        out_type=jax.ShapeDtypeStruct((_NC * n, d), jnp.float32),
        mesh=mesh,
        scratch_types=[
            pltpu.VMEM_SHARED((n, d), jnp.float32),
            pltpu.VMEM((epw,), jnp.int32),
            pltpu.VMEM((epw,), jnp.int32),
            [pltpu.VMEM((_K, d), jnp.float32) for _ in range(_NB)],
            [pltpu.SemaphoreType.DMA for _ in range(_NB)],
            [pltpu.SemaphoreType.DMA for _ in range(_NB)],
        ],
    )
    return run(msg, ref_a, ref_b, zeros)


def _hw_body(x_ref, grk_ref, gb1_ref, hw_ref):
    hw_ref[...] = (jnp.dot(x_ref[...], grk_ref[...],
                           preferred_element_type=jnp.float32) + gb1_ref[...])


def _hw_call(X, grk, gb1, block_n):
    n, d = X.shape
    u3 = grk.shape[1]
    return pl.pallas_call(
        _hw_body,
        grid=(n // block_n,),
        in_specs=[
            pl.BlockSpec((block_n, d), lambda i: (i, 0)),
            pl.BlockSpec(grk.shape, lambda i: (0, 0)),
            pl.BlockSpec(gb1.shape, lambda i: (0, 0)),
        ],
        out_specs=pl.BlockSpec((block_n, u3), lambda i: (i, 0)),
        out_shape=jax.ShapeDtypeStruct((n, u3), jnp.float32),
    )(X, grk, gb1)


def _gru_body(a0_ref, a1_ref, x_ref, gk_ref, hw_ref, gb0_ref, out_ref):
    u = x_ref.shape[1]
    agg = a0_ref[...] + a1_ref[...]
    x = x_ref[...]
    xw = jnp.dot(agg, gk_ref[...], preferred_element_type=jnp.float32) + gb0_ref[...]
    hw = hw_ref[...]
    x_z, x_r, x_h = xw[:, :u], xw[:, u:2 * u], xw[:, 2 * u:]
    h_z, h_r, h_h = hw[:, :u], hw[:, u:2 * u], hw[:, 2 * u:]
    z = jax.nn.sigmoid(x_z + h_z)
    r = jax.nn.sigmoid(x_r + h_r)
    hh = jnp.tanh(x_h + r * h_h)
    out_ref[...] = z * x + (1.0 - z) * hh


def _gru_call(partials, X, gk, hw, gb0, block_n):
    n, d = X.shape
    u3 = gk.shape[1]
    goff = n // block_n
    return pl.pallas_call(
        _gru_body,
        grid=(goff,),
        in_specs=[
            pl.BlockSpec((block_n, d), lambda i: (i, 0)),
            pl.BlockSpec((block_n, d), lambda i, goff=goff: (goff + i, 0)),
            pl.BlockSpec((block_n, d), lambda i: (i, 0)),
            pl.BlockSpec(gk.shape, lambda i: (0, 0)),
            pl.BlockSpec((block_n, u3), lambda i: (i, 0)),
            pl.BlockSpec(gb0.shape, lambda i: (0, 0)),
        ],
        out_specs=pl.BlockSpec((block_n, d), lambda i: (i, 0)),
        out_shape=jax.ShapeDtypeStruct((n, d), jnp.float32),
    )(partials, partials, X, gk, hw, gb0)


def kernel(X, ref_a, ref_b, W0, b0, W1, b1, gru_kernel, gru_recurrent_kernel,
           gru_bias):
    n, d = X.shape
    u = W0.shape[1]
    block_n = 2000
    msg = _dense_call(X, W0, b0.reshape(1, u), W1, b1.reshape(1, u), block_n)
    zeros = jnp.zeros((n, d), jnp.float32)
    partials = _sc_call(msg, ref_a, ref_b, zeros)
    hw = _hw_call(X, gru_recurrent_kernel, gru_bias[1].reshape(1, -1),
                  block_n)
    return _gru_call(partials, X, gru_kernel, hw,
                     gru_bias[0].reshape(1, -1), block_n)


# speedup vs baseline: 1.0263x; 1.0263x over previous
import functools

import jax
import jax.numpy as jnp
from jax import lax
from jax.experimental import pallas as pl
from jax.experimental.pallas import tpu as pltpu
from jax.experimental.pallas import tpu_sc as plsc

_NC = 2
_NS = 16
_K = 40
_NB = 6


def _dense_body(x_ref, w0_ref, b0_ref, w1_ref, b1_ref, msg_ref):
    x = x_ref[...]
    h = jnp.dot(x, w0_ref[...], preferred_element_type=jnp.float32) + b0_ref[...]
    msg_ref[...] = jnp.dot(h, w1_ref[...], preferred_element_type=jnp.float32) + b1_ref[...]


def _dense_call(X, W0, b0, W1, b1, block_n):
    n, d = X.shape
    grid = n // block_n
    return pl.pallas_call(
        _dense_body,
        grid=(grid,),
        in_specs=[
            pl.BlockSpec((block_n, d), lambda i: (i, 0)),
            pl.BlockSpec(W0.shape, lambda i: (0, 0)),
            pl.BlockSpec(b0.shape, lambda i: (0, 0)),
            pl.BlockSpec(W1.shape, lambda i: (0, 0)),
            pl.BlockSpec(b1.shape, lambda i: (0, 0)),
        ],
        out_specs=pl.BlockSpec((block_n, d), lambda i: (i, 0)),
        out_shape=jax.ShapeDtypeStruct((n, d), jnp.float32),
    )(X, W0, b0, W1, b1)


def _sc_body(n, e, d, nchunk, xmsg_hbm, ra1_hbm, rb1_hbm,
             zeros_hbm, out_hbm, acc, gidx, sidx, rows, gsems, ssems):
    epw = nchunk * _K
    rpt = (n // _NS) // 8 * 8
    rem = n - _NS * rpt
    c = lax.axis_index("c")
    s = lax.axis_index("s")
    w = c * _NS + s
    r0 = s * rpt
    sets = tuple((rows[p], gsems[p], ssems[p]) for p in range(_NB))

    def one_direction(gi, si):
        def gather(i, p):
            row, gsem, _ = sets[p]
            pltpu.async_copy(xmsg_hbm.at[gi.at[pl.ds(i * _K, _K)]], row,
                             gsem)

        def wait_gather(i, p):
            row, gsem, _ = sets[p]
            pltpu.make_async_copy(xmsg_hbm.at[gi.at[pl.ds(i * _K, _K)]],
                                  row, gsem).wait()

        def scatter(i, p):
            row, _, ssem = sets[p]
            pltpu.async_copy(row, acc.at[si.at[pl.ds(i * _K, _K)]], ssem,
                             add=True)

        def wait_scatter(i, p):
            row, _, ssem = sets[p]
            pltpu.make_async_copy(row, acc.at[si.at[pl.ds(i * _K, _K)]],
                                  ssem).wait()

        def step(i, p, prefetch, wait_prev=True):
            wait_gather(i, p)
            scatter(i, p)
            if prefetch:
                pm1 = (p + _NB - 1) % _NB
                if wait_prev:
                    wait_scatter(i - 1, pm1)
                gather(i + _NB - 1, pm1)

        for j in range(_NB - 1):
            gather(j, j)
        for i in range(_NB - 1):
            step(i, i, True, wait_prev=(i >= 1))
        lo = _NB - 1
        hi = nchunk - _NB
        iters = (hi - lo + 1) // _NB

        def block(t, carry):
            i0 = lo + _NB * t
            for k in range(_NB):
                step(i0 + k, (lo + k) % _NB, True)
            return carry

        lax.fori_loop(0, iters, block, 0)
        for i in range(lo + iters * _NB, hi + 1):
            step(i, i % _NB, True)
        for i in range(hi + 1, nchunk):
            step(i, i % _NB, False)
        for j in range(nchunk - _NB, nchunk):
            wait_scatter(j, j % _NB)

    psem = gsems[0]
    cz = pltpu.async_copy(zeros_hbm.at[pl.ds(r0, rpt)],
                          acc.at[pl.ds(r0, rpt)], psem)
    ca = pltpu.async_copy(ra1_hbm.at[pl.ds(w * epw, epw)], gidx, psem)
    cb = pltpu.async_copy(rb1_hbm.at[pl.ds(w * epw, epw)], sidx, psem)
    if rem:
        @pl.when(s == _NS - 1)
        def _zero_rem():
            pltpu.sync_copy(zeros_hbm.at[pl.ds(_NS * rpt, rem)],
                            acc.at[pl.ds(_NS * rpt, rem)])
    cz.wait()
    ca.wait()
    cb.wait()
    plsc.subcore_barrier()

    one_direction(gidx, sidx)
    one_direction(sidx, gidx)

    plsc.subcore_barrier()
    pltpu.sync_copy(acc.at[pl.ds(r0, rpt)], out_hbm.at[pl.ds(c * n + r0, rpt)])
    if rem:
        @pl.when(s == _NS - 1)
        def _out_rem():
            pltpu.sync_copy(acc.at[pl.ds(_NS * rpt, rem)],
                            out_hbm.at[pl.ds(c * n + _NS * rpt, rem)])


def _sc_call(msg, ref_a, ref_b, zeros):
    n, d = msg.shape
    e = ref_a.shape[0]
    nw = _NC * _NS
    nchunk = e // (nw * _K)
    epw = nchunk * _K
    mesh = plsc.VectorSubcoreMesh(core_axis_name="c", subcore_axis_name="s")
    run = pl.kernel(
        functools.partial(_sc_body, n, e, d, nchunk),
        out_type=jax.ShapeDtypeStruct((_NC * n, d), jnp.float32),
        mesh=mesh,
        scratch_types=[
            pltpu.VMEM_SHARED((n, d), jnp.float32),
            pltpu.VMEM((epw,), jnp.int32),
            pltpu.VMEM((epw,), jnp.int32),
            [pltpu.VMEM((_K, d), jnp.float32) for _ in range(_NB)],
            [pltpu.SemaphoreType.DMA for _ in range(_NB)],
            [pltpu.SemaphoreType.DMA for _ in range(_NB)],
        ],
    )
    return run(msg, ref_a, ref_b, zeros)


def _gru_body(a0_ref, a1_ref, x_ref, gk_ref, grk_ref, gb0_ref, gb1_ref,
              out_ref):
    u = x_ref.shape[1]
    agg = a0_ref[...] + a1_ref[...]
    x = x_ref[...]
    xw = jnp.dot(agg, gk_ref[...], preferred_element_type=jnp.float32) + gb0_ref[...]
    hw = jnp.dot(x, grk_ref[...], preferred_element_type=jnp.float32) + gb1_ref[...]
    x_z, x_r, x_h = xw[:, :u], xw[:, u:2 * u], xw[:, 2 * u:]
    h_z, h_r, h_h = hw[:, :u], hw[:, u:2 * u], hw[:, 2 * u:]
    z = jax.nn.sigmoid(x_z + h_z)
    r = jax.nn.sigmoid(x_r + h_r)
    hh = jnp.tanh(x_h + r * h_h)
    out_ref[...] = z * x + (1.0 - z) * hh


def _gru_call(partials, X, gk, grk, gb0, gb1, block_n):
    n, d = X.shape
    goff = n // block_n
    return pl.pallas_call(
        _gru_body,
        grid=(goff,),
        in_specs=[
            pl.BlockSpec((block_n, d), lambda i: (i, 0)),
            pl.BlockSpec((block_n, d), lambda i, goff=goff: (goff + i, 0)),
            pl.BlockSpec((block_n, d), lambda i: (i, 0)),
            pl.BlockSpec(gk.shape, lambda i: (0, 0)),
            pl.BlockSpec(grk.shape, lambda i: (0, 0)),
            pl.BlockSpec(gb0.shape, lambda i: (0, 0)),
            pl.BlockSpec(gb1.shape, lambda i: (0, 0)),
        ],
        out_specs=pl.BlockSpec((block_n, d), lambda i: (i, 0)),
        out_shape=jax.ShapeDtypeStruct((n, d), jnp.float32),
    )(partials, partials, X, gk, grk, gb0, gb1)


def kernel(X, ref_a, ref_b, W0, b0, W1, b1, gru_kernel, gru_recurrent_kernel,
           gru_bias):
    n, d = X.shape
    u = W0.shape[1]
    block_n = 2000
    msg = _dense_call(X, W0, b0.reshape(1, u), W1, b1.reshape(1, u), block_n)
    zeros = jnp.zeros((n, d), jnp.float32)
    partials = _sc_call(msg, ref_a, ref_b, zeros)
    return _gru_call(partials, X, gru_kernel, gru_recurrent_kernel,
                     gru_bias[0].reshape(1, -1), gru_bias[1].reshape(1, -1),
                     block_n)


# speedup vs baseline: 1.0356x; 1.0091x over previous
import functools

import jax
import jax.numpy as jnp
from jax import lax
from jax.experimental import pallas as pl
from jax.experimental.pallas import tpu as pltpu
from jax.experimental.pallas import tpu_sc as plsc

_NC = 2
_NS = 16
_K = 40
_NB = 6


def _dense_body(x_ref, w0_ref, b0_ref, w1_ref, b1_ref, msg_ref):
    x = x_ref[...]
    h = jnp.dot(x, w0_ref[...], preferred_element_type=jnp.float32) + b0_ref[...]
    msg_ref[...] = jnp.dot(h, w1_ref[...], preferred_element_type=jnp.float32) + b1_ref[...]


def _dense_call(X, W0, b0, W1, b1, block_n):
    n, d = X.shape
    grid = n // block_n
    return pl.pallas_call(
        _dense_body,
        grid=(grid,),
        in_specs=[
            pl.BlockSpec((block_n, d), lambda i: (i, 0)),
            pl.BlockSpec(W0.shape, lambda i: (0, 0)),
            pl.BlockSpec(b0.shape, lambda i: (0, 0)),
            pl.BlockSpec(W1.shape, lambda i: (0, 0)),
            pl.BlockSpec(b1.shape, lambda i: (0, 0)),
        ],
        out_specs=pl.BlockSpec((block_n, d), lambda i: (i, 0)),
        out_shape=jax.ShapeDtypeStruct((n, d), jnp.float32),
    )(X, W0, b0, W1, b1)


def _sc_body(n, e, d, nchunk, xmsg_hbm, ra1_hbm, rb1_hbm,
             zeros_hbm, out_hbm, acc, gidx, sidx, rows, gsems, ssems):
    epw = nchunk * _K
    rpt = (n // _NS) // 8 * 8
    rem = n - _NS * rpt
    c = lax.axis_index("c")
    s = lax.axis_index("s")
    w = c * _NS + s
    r0 = s * rpt
    sets = tuple((rows[p], gsems[p], ssems[p]) for p in range(_NB))

    def run_both():
        def refs(j):
            if j < nchunk:
                return gidx, sidx, j
            return sidx, gidx, j - nchunk

        def gather(gi, i, p):
            row, gsem, _ = sets[p]
            pltpu.async_copy(xmsg_hbm.at[gi.at[pl.ds(i * _K, _K)]], row,
                             gsem)

        def wait_gather(gi, i, p):
            row, gsem, _ = sets[p]
            pltpu.make_async_copy(xmsg_hbm.at[gi.at[pl.ds(i * _K, _K)]],
                                  row, gsem).wait()

        def scatter(si, i, p):
            row, _, ssem = sets[p]
            pltpu.async_copy(row, acc.at[si.at[pl.ds(i * _K, _K)]], ssem,
                             add=True)

        def wait_scatter(si, i, p):
            row, _, ssem = sets[p]
            pltpu.make_async_copy(row, acc.at[si.at[pl.ds(i * _K, _K)]],
                                  ssem).wait()

        def step_static(j, prefetch=True, wait_prev=True):
            gi, si, i = refs(j)
            wait_gather(gi, i, j % _NB)
            scatter(si, i, j % _NB)
            if prefetch and j + _NB - 1 <= 2 * nchunk - 1:
                pm1 = (j + _NB - 1) % _NB
                if wait_prev:
                    gi1, si1, i1 = refs(j - 1)
                    wait_scatter(si1, i1, pm1)
                gi2, si2, i2 = refs(j + _NB - 1)
                gather(gi2, i2, pm1)

        def block_loop(lo, hi, gi, si, off):
            iters = (hi - lo + 1) // _NB

            def block(t, carry):
                j0 = lo + _NB * t
                for k in range(_NB):
                    p = (lo + k) % _NB
                    i = j0 + k - off
                    wait_gather(gi, i, p)
                    scatter(si, i, p)
                    wait_scatter(si, i - 1, (p + _NB - 1) % _NB)
                    gather(gi, i + _NB - 1, (p + _NB - 1) % _NB)
                return carry

            lax.fori_loop(0, iters, block, 0)
            return lo + iters * _NB

        for j in range(_NB - 1):
            gi, si, i = refs(j)
            gather(gi, i, j % _NB)
        for j in range(_NB - 1):
            step_static(j, wait_prev=(j >= 1))
        nxt = block_loop(_NB - 1, nchunk - _NB, gidx, sidx, 0)
        for j in range(nxt, nchunk + _NB - 1):
            step_static(j)
        nxt = block_loop(nchunk + _NB - 1, 2 * nchunk - _NB, sidx, gidx,
                         nchunk)
        for j in range(nxt, 2 * nchunk - _NB + 1):
            step_static(j)
        for j in range(2 * nchunk - _NB + 1, 2 * nchunk):
            step_static(j, prefetch=False)
        for j in range(2 * nchunk - _NB, 2 * nchunk):
            gi, si, i = refs(j)
            wait_scatter(si, i, j % _NB)

    psem = gsems[0]
    cz = pltpu.async_copy(zeros_hbm.at[pl.ds(r0, rpt)],
                          acc.at[pl.ds(r0, rpt)], psem)
    ca = pltpu.async_copy(ra1_hbm.at[pl.ds(w * epw, epw)], gidx, psem)
    cb = pltpu.async_copy(rb1_hbm.at[pl.ds(w * epw, epw)], sidx, psem)
    if rem:
        @pl.when(s == _NS - 1)
        def _zero_rem():
            pltpu.sync_copy(zeros_hbm.at[pl.ds(_NS * rpt, rem)],
                            acc.at[pl.ds(_NS * rpt, rem)])
    cz.wait()
    ca.wait()
    cb.wait()
    plsc.subcore_barrier()

    run_both()

    plsc.subcore_barrier()
    pltpu.sync_copy(acc.at[pl.ds(r0, rpt)], out_hbm.at[pl.ds(c * n + r0, rpt)])
    if rem:
        @pl.when(s == _NS - 1)
        def _out_rem():
            pltpu.sync_copy(acc.at[pl.ds(_NS * rpt, rem)],
                            out_hbm.at[pl.ds(c * n + _NS * rpt, rem)])


def _sc_call(msg, ref_a, ref_b, zeros):
    n, d = msg.shape
    e = ref_a.shape[0]
    nw = _NC * _NS
    nchunk = e // (nw * _K)
    epw = nchunk * _K
    mesh = plsc.VectorSubcoreMesh(core_axis_name="c", subcore_axis_name="s")
    run = pl.kernel(
        functools.partial(_sc_body, n, e, d, nchunk),
        out_type=jax.ShapeDtypeStruct((_NC * n, d), jnp.float32),
        mesh=mesh,
        scratch_types=[
            pltpu.VMEM_SHARED((n, d), jnp.float32),
            pltpu.VMEM((epw,), jnp.int32),
            pltpu.VMEM((epw,), jnp.int32),
            [pltpu.VMEM((_K, d), jnp.float32) for _ in range(_NB)],
            [pltpu.SemaphoreType.DMA for _ in range(_NB)],
            [pltpu.SemaphoreType.DMA for _ in range(_NB)],
        ],
    )
    return run(msg, ref_a, ref_b, zeros)


def _gru_body(a0_ref, a1_ref, x_ref, gk_ref, grk_ref, gb0_ref, gb1_ref,
              out_ref):
    u = x_ref.shape[1]
    agg = a0_ref[...] + a1_ref[...]
    x = x_ref[...]
    xw = jnp.dot(agg, gk_ref[...], preferred_element_type=jnp.float32) + gb0_ref[...]
    hw = jnp.dot(x, grk_ref[...], preferred_element_type=jnp.float32) + gb1_ref[...]
    x_z, x_r, x_h = xw[:, :u], xw[:, u:2 * u], xw[:, 2 * u:]
    h_z, h_r, h_h = hw[:, :u], hw[:, u:2 * u], hw[:, 2 * u:]
    z = jax.nn.sigmoid(x_z + h_z)
    r = jax.nn.sigmoid(x_r + h_r)
    hh = jnp.tanh(x_h + r * h_h)
    out_ref[...] = z * x + (1.0 - z) * hh


def _gru_call(partials, X, gk, grk, gb0, gb1, block_n):
    n, d = X.shape
    goff = n // block_n
    return pl.pallas_call(
        _gru_body,
        grid=(goff,),
        in_specs=[
            pl.BlockSpec((block_n, d), lambda i: (i, 0)),
            pl.BlockSpec((block_n, d), lambda i, goff=goff: (goff + i, 0)),
            pl.BlockSpec((block_n, d), lambda i: (i, 0)),
            pl.BlockSpec(gk.shape, lambda i: (0, 0)),
            pl.BlockSpec(grk.shape, lambda i: (0, 0)),
            pl.BlockSpec(gb0.shape, lambda i: (0, 0)),
            pl.BlockSpec(gb1.shape, lambda i: (0, 0)),
        ],
        out_specs=pl.BlockSpec((block_n, d), lambda i: (i, 0)),
        out_shape=jax.ShapeDtypeStruct((n, d), jnp.float32),
    )(partials, partials, X, gk, grk, gb0, gb1)


def kernel(X, ref_a, ref_b, W0, b0, W1, b1, gru_kernel, gru_recurrent_kernel,
           gru_bias):
    n, d = X.shape
    u = W0.shape[1]
    block_n = 2000
    msg = _dense_call(X, W0, b0.reshape(1, u), W1, b1.reshape(1, u), block_n)
    zeros = jnp.zeros((n, d), jnp.float32)
    partials = _sc_call(msg, ref_a, ref_b, zeros)
    return _gru_call(partials, X, gru_kernel, gru_recurrent_kernel,
                     gru_bias[0].reshape(1, -1), gru_bias[1].reshape(1, -1),
                     block_n)
